# baseline (speedup 1.0000x reference)
import functools

import jax
import jax.numpy as jnp
from jax import lax
from jax.experimental import pallas as pl
from jax.experimental.pallas import tpu as pltpu
from jax.experimental.pallas import tpu_sc as plsc

NUM_BILLS = 1000000
NUM_LEGS = 100000
BILL_STRIDE = 1007616
LEG_STRIDE = 102400
BILL_BLK = BILL_STRIDE // 24
LEG_BLK = LEG_STRIDE // 4

BATCH = 16384
LATENT_DIM = 16
NUM_WORKERS = 32
PAIRS_PER_WORKER = BATCH // NUM_WORKERS
CHUNK = 128
CHUNKS_PER_WORKER = PAIRS_PER_WORKER // CHUNK
GROUPS = PAIRS_PER_WORKER // 16


def _flatten_body(t_ref, out_ref):
    out_ref[...] = t_ref[pl.program_id(1), :]


def _dim_major_flat(table_t, blk, blocks_per_row, stride):
    return pl.pallas_call(
        _flatten_body,
        grid=(blocks_per_row, LATENT_DIM),
        in_specs=[pl.BlockSpec((LATENT_DIM, blk), lambda j, d: (0, j))],
        out_specs=pl.BlockSpec(
            (blk,), lambda j, d: (d * blocks_per_row + j,)),
        out_shape=jax.ShapeDtypeStruct((LATENT_DIM * stride,), jnp.float32),
    )(table_t)


def _sc_body(bids, lids, gb, leg_b, bill_b, leg_t, bill_t, out_hbm,
             bidx, lidx, bgidx, lgidx, bcols, lcols, bb_v, lb_v, gb_v,
             out_v, sem, sem2):
    wid = lax.axis_index("s") * 2 + lax.axis_index("c")
    base = wid * PAIRS_PER_WORKER

    pltpu.sync_copy(bids.at[pl.ds(base, PAIRS_PER_WORKER)], bidx)
    pltpu.sync_copy(lids.at[pl.ds(base, PAIRS_PER_WORKER)], lidx)
    pltpu.sync_copy(gb, gb_v)

    bias_copies = []
    for c in range(CHUNKS_PER_WORKER):
        sl = pl.ds(c * CHUNK, CHUNK)
        bias_copies.append(pltpu.async_copy(
            bill_b.at[bidx.at[sl]], bb_v.at[sl], sem))
        bias_copies.append(pltpu.async_copy(
            leg_b.at[lidx.at[sl]], lb_v.at[sl], sem))

    def build(v, _):
        sl = pl.ds(v * 16, 16)
        bv = bidx[sl]
        lv = lidx[sl]
        for d in range(LATENT_DIM):
            bgidx[d, sl] = bv + d * BILL_STRIDE
            lgidx[d, sl] = lv + d * LEG_STRIDE
        return 0

    lax.fori_loop(0, GROUPS, build, 0, unroll=False)

    for cp in bias_copies:
        cp.wait()

    def fire(d, s):
        copies = []
        for c in range(CHUNKS_PER_WORKER):
            sl = pl.ds(c * CHUNK, CHUNK)
            copies.append(pltpu.async_copy(
                bill_t.at[bgidx.at[d, sl]], bcols.at[d, sl], s))
            copies.append(pltpu.async_copy(
                leg_t.at[lgidx.at[d, sl]], lcols.at[d, sl], s))
        return copies

    def gather_pair(i, _):
        d0 = i * 2
        c0 = fire(d0, sem)
        c1 = fire(d0 + 1, sem2)
        for cp in c0:
            cp.wait()
        for cp in c1:
            cp.wait()
        return 0

    lax.fori_loop(0, LATENT_DIM // 2, gather_pair, 0, unroll=False)

    gbv = gb_v[...]

    def group(g, _):
        sl = pl.ds(g * 16, 16)
        acc = bcols[0, sl] * lcols[0, sl]
        for d in range(1, LATENT_DIM):
            acc = acc + bcols[d, sl] * lcols[d, sl]
        x = gbv + bb_v[sl] + lb_v[sl] + acc
        out_v[sl] = 1.0 / (1.0 + jnp.exp(-x))
        return 0

    lax.fori_loop(0, GROUPS, group, 0, unroll=False)

    pltpu.sync_copy(out_v, out_hbm.at[pl.ds(base, PAIRS_PER_WORKER)])


@jax.jit
def _predict(bids, lids, gb, leg_b, bill_b, leg_emb_t, bill_emb_t):
    leg_t = _dim_major_flat(leg_emb_t, LEG_BLK, 4, LEG_STRIDE)
    bill_t = _dim_major_flat(bill_emb_t, BILL_BLK, 24, BILL_STRIDE)

    mesh = plsc.VectorSubcoreMesh(core_axis_name="c", subcore_axis_name="s")
    k = pl.kernel(
        _sc_body,
        out_type=jax.ShapeDtypeStruct((BATCH,), jnp.float32),
        mesh=mesh,
        compiler_params=pltpu.CompilerParams(needs_layout_passes=False,
                                             use_tc_tiling_on_sc=False),
        scratch_types=[
            pltpu.VMEM((PAIRS_PER_WORKER,), jnp.int32),
            pltpu.VMEM((PAIRS_PER_WORKER,), jnp.int32),
            pltpu.VMEM((LATENT_DIM, PAIRS_PER_WORKER), jnp.int32),
            pltpu.VMEM((LATENT_DIM, PAIRS_PER_WORKER), jnp.int32),
            pltpu.VMEM((LATENT_DIM, PAIRS_PER_WORKER), jnp.float32),
            pltpu.VMEM((LATENT_DIM, PAIRS_PER_WORKER), jnp.float32),
            pltpu.VMEM((PAIRS_PER_WORKER,), jnp.float32),
            pltpu.VMEM((PAIRS_PER_WORKER,), jnp.float32),
            pltpu.VMEM((16,), jnp.float32),
            pltpu.VMEM((PAIRS_PER_WORKER,), jnp.float32),
            pltpu.SemaphoreType.DMA,
            pltpu.SemaphoreType.DMA,
        ],
    )
    return k(bids, lids, gb, leg_b, bill_b, leg_t, bill_t)


def kernel(bill_ids, legislator_ids, global_bias, legislator_bias, bill_bias,
           legislator_embedding, bill_embedding):
    bids = bill_ids.astype(jnp.int32)
    lids = legislator_ids.astype(jnp.int32)
    gb = jnp.broadcast_to(jnp.reshape(global_bias, (1,)), (16,))
    leg_b = jnp.reshape(legislator_bias, (-1,))
    bill_b = jnp.reshape(bill_bias, (-1,))
    leg_emb_t = jnp.transpose(legislator_embedding)
    bill_emb_t = jnp.transpose(bill_embedding)
    return _predict(bids, lids, gb, leg_b, bill_b, leg_emb_t, bill_emb_t)


# speedup vs baseline: 1.8936x; 1.8936x over previous
import jax
import jax.numpy as jnp
from jax import lax
from jax.experimental import pallas as pl
from jax.experimental.pallas import tpu as pltpu
from jax.experimental.pallas import tpu_sc as plsc

NUM_BILLS = 1000000
NUM_LEGS = 100000
BILL_STRIDE = 1007616
LEG_STRIDE = 102400
BILL_BLK = BILL_STRIDE // 8
LEG_BLK = LEG_STRIDE // 2

BATCH = 16384
LATENT_DIM = 16
ROWS = LATENT_DIM + 1
NUM_WORKERS = 32
PAIRS_PER_WORKER = BATCH // NUM_WORKERS
CHUNK = 128
CHUNKS_PER_WORKER = PAIRS_PER_WORKER // CHUNK
GROUPS = PAIRS_PER_WORKER // 16


def _flatten_body(t_ref, b_ref, out_ref):
    d = pl.program_id(1)

    @pl.when(d < LATENT_DIM)
    def _():
        out_ref[...] = t_ref[d, :]

    @pl.when(d == LATENT_DIM)
    def _():
        out_ref[...] = b_ref[0, :]


def _dim_major_flat(table_t, bias_t, blk, blocks_per_row, stride):
    return pl.pallas_call(
        _flatten_body,
        grid=(blocks_per_row, ROWS),
        in_specs=[
            pl.BlockSpec((LATENT_DIM, blk), lambda j, d: (0, j)),
            pl.BlockSpec((1, blk), lambda j, d: (0, j)),
        ],
        out_specs=pl.BlockSpec(
            (blk,), lambda j, d: (d * blocks_per_row + j,)),
        out_shape=jax.ShapeDtypeStruct((ROWS * stride,), jnp.float32),
    )(table_t, bias_t)


def _sc_body(bids, lids, gb, leg_t, bill_t, out_hbm,
             bidx, lidx, bgidx, lgidx, bcols, lcols, bb_v, lb_v, gb_v,
             out_v, sem, sem2, sem3):
    wid = lax.axis_index("s") * 2 + lax.axis_index("c")
    base = wid * PAIRS_PER_WORKER

    pltpu.sync_copy(bids.at[pl.ds(base, PAIRS_PER_WORKER)], bidx)
    pltpu.sync_copy(lids.at[pl.ds(base, PAIRS_PER_WORKER)], lidx)
    pltpu.sync_copy(gb, gb_v)

    def build(v, _):
        sl = pl.ds(v * 16, 16)
        bv = bidx[sl]
        lv = lidx[sl]
        for d in range(ROWS):
            bgidx[d, sl] = bv + d * BILL_STRIDE
            lgidx[d, sl] = lv + d * LEG_STRIDE
        return 0

    lax.fori_loop(0, GROUPS, build, 0, unroll=False)

    bias_copies = []
    for c in range(CHUNKS_PER_WORKER):
        sl = pl.ds(c * CHUNK, CHUNK)
        bias_copies.append(pltpu.async_copy(
            bill_t.at[bgidx.at[LATENT_DIM, sl]], bb_v.at[sl], sem3))
        bias_copies.append(pltpu.async_copy(
            leg_t.at[lgidx.at[LATENT_DIM, sl]], lb_v.at[sl], sem3))

    def fire(d, s):
        copies = []
        for c in range(CHUNKS_PER_WORKER):
            sl = pl.ds(c * CHUNK, CHUNK)
            copies.append(pltpu.async_copy(
                bill_t.at[bgidx.at[d, sl]], bcols.at[d, sl], s))
            copies.append(pltpu.async_copy(
                leg_t.at[lgidx.at[d, sl]], lcols.at[d, sl], s))
        return copies

    def gather_pair(i, _):
        d0 = i * 2
        c0 = fire(d0, sem)
        c1 = fire(d0 + 1, sem2)
        for cp in c0:
            cp.wait()
        for cp in c1:
            cp.wait()
        return 0

    lax.fori_loop(0, LATENT_DIM // 2, gather_pair, 0, unroll=False)

    for cp in bias_copies:
        cp.wait()

    gbv = gb_v[...]

    def group(g, _):
        sl = pl.ds(g * 16, 16)
        acc = bcols[0, sl] * lcols[0, sl]
        for d in range(1, LATENT_DIM):
            acc = acc + bcols[d, sl] * lcols[d, sl]
        x = gbv + bb_v[sl] + lb_v[sl] + acc
        out_v[sl] = 1.0 / (1.0 + jnp.exp(-x))
        return 0

    lax.fori_loop(0, GROUPS, group, 0, unroll=False)

    pltpu.sync_copy(out_v, out_hbm.at[pl.ds(base, PAIRS_PER_WORKER)])


@jax.jit
def _predict(bids, lids, gb, leg_bias_t, bill_bias_t, leg_emb_t, bill_emb_t):
    leg_t = _dim_major_flat(leg_emb_t, leg_bias_t, LEG_BLK, 2, LEG_STRIDE)
    bill_t = _dim_major_flat(bill_emb_t, bill_bias_t, BILL_BLK, 8,
                             BILL_STRIDE)

    mesh = plsc.VectorSubcoreMesh(core_axis_name="c", subcore_axis_name="s")
    k = pl.kernel(
        _sc_body,
        out_type=jax.ShapeDtypeStruct((BATCH,), jnp.float32),
        mesh=mesh,
        compiler_params=pltpu.CompilerParams(needs_layout_passes=False,
                                             use_tc_tiling_on_sc=False),
        scratch_types=[
            pltpu.VMEM((PAIRS_PER_WORKER,), jnp.int32),
            pltpu.VMEM((PAIRS_PER_WORKER,), jnp.int32),
            pltpu.VMEM((ROWS, PAIRS_PER_WORKER), jnp.int32),
            pltpu.VMEM((ROWS, PAIRS_PER_WORKER), jnp.int32),
            pltpu.VMEM((LATENT_DIM, PAIRS_PER_WORKER), jnp.float32),
            pltpu.VMEM((LATENT_DIM, PAIRS_PER_WORKER), jnp.float32),
            pltpu.VMEM((PAIRS_PER_WORKER,), jnp.float32),
            pltpu.VMEM((PAIRS_PER_WORKER,), jnp.float32),
            pltpu.VMEM((16,), jnp.float32),
            pltpu.VMEM((PAIRS_PER_WORKER,), jnp.float32),
            pltpu.SemaphoreType.DMA,
            pltpu.SemaphoreType.DMA,
            pltpu.SemaphoreType.DMA,
        ],
    )
    return k(bids, lids, gb, leg_t, bill_t)


def kernel(bill_ids, legislator_ids, global_bias, legislator_bias, bill_bias,
           legislator_embedding, bill_embedding):
    bids = bill_ids.astype(jnp.int32)
    lids = legislator_ids.astype(jnp.int32)
    gb = jnp.broadcast_to(jnp.reshape(global_bias, (1,)), (16,))
    leg_bias_t = jnp.transpose(legislator_bias)
    bill_bias_t = jnp.transpose(bill_bias)
    leg_emb_t = jnp.transpose(legislator_embedding)
    bill_emb_t = jnp.transpose(bill_embedding)
    return _predict(bids, lids, gb, leg_bias_t, bill_bias_t,
                    leg_emb_t, bill_emb_t)


# speedup vs baseline: 2.3358x; 1.2335x over previous
import jax
import jax.numpy as jnp
from jax import lax
from jax.experimental import pallas as pl
from jax.experimental.pallas import tpu as pltpu
from jax.experimental.pallas import tpu_sc as plsc

NUM_BILLS = 1000000
NUM_LEGS = 100000
BILL_STRIDE = 1007616
LEG_STRIDE = 102400
BILL_BLK = BILL_STRIDE // 4
LEG_BLK = LEG_STRIDE // 1

BATCH = 16384
LATENT_DIM = 16
ROWS = LATENT_DIM + 1
NUM_WORKERS = 32
PAIRS_PER_WORKER = BATCH // NUM_WORKERS
CHUNK = 128
CHUNKS_PER_WORKER = PAIRS_PER_WORKER // CHUNK
GROUPS = PAIRS_PER_WORKER // 16


def _flatten_body(t_ref, b_ref, out_ref):
    d = pl.program_id(1)

    @pl.when(d < LATENT_DIM)
    def _():
        out_ref[...] = t_ref[d, :]

    @pl.when(d == LATENT_DIM)
    def _():
        out_ref[...] = b_ref[0, :]


def _dim_major_flat(table_t, bias_t, blk, blocks_per_row, stride):
    return pl.pallas_call(
        _flatten_body,
        grid=(blocks_per_row, ROWS),
        in_specs=[
            pl.BlockSpec((LATENT_DIM, blk), lambda j, d: (0, j)),
            pl.BlockSpec((1, blk), lambda j, d: (0, j)),
        ],
        out_specs=pl.BlockSpec(
            (blk,), lambda j, d: (d * blocks_per_row + j,)),
        out_shape=jax.ShapeDtypeStruct((ROWS * stride,), jnp.float32),
    )(table_t, bias_t)


def _sc_body(bids, lids, gb, leg_t, bill_t, out_hbm,
             bidx, lidx, bgidx, lgidx, bcols, lcols, bb_v, lb_v, gb_v,
             out_v, sem, sem2, sem3):
    wid = lax.axis_index("s") * 2 + lax.axis_index("c")
    base = wid * PAIRS_PER_WORKER

    pltpu.sync_copy(bids.at[pl.ds(base, PAIRS_PER_WORKER)], bidx)
    pltpu.sync_copy(lids.at[pl.ds(base, PAIRS_PER_WORKER)], lidx)
    pltpu.sync_copy(gb, gb_v)

    def build(v, _):
        sl = pl.ds(v * 16, 16)
        bv = bidx[sl]
        lv = lidx[sl]
        for d in range(ROWS):
            bgidx[d, sl] = bv + d * BILL_STRIDE
            lgidx[d, sl] = lv + d * LEG_STRIDE
        return 0

    lax.fori_loop(0, GROUPS, build, 0, unroll=False)

    bias_copies = []
    for c in range(CHUNKS_PER_WORKER):
        sl = pl.ds(c * CHUNK, CHUNK)
        bias_copies.append(pltpu.async_copy(
            bill_t.at[bgidx.at[LATENT_DIM, sl]], bb_v.at[sl], sem3))
        bias_copies.append(pltpu.async_copy(
            leg_t.at[lgidx.at[LATENT_DIM, sl]], lb_v.at[sl], sem3))

    def fire(d, s):
        copies = []
        for c in range(CHUNKS_PER_WORKER):
            sl = pl.ds(c * CHUNK, CHUNK)
            copies.append(pltpu.async_copy(
                bill_t.at[bgidx.at[d, sl]], bcols.at[d, sl], s))
            copies.append(pltpu.async_copy(
                leg_t.at[lgidx.at[d, sl]], lcols.at[d, sl], s))
        return copies

    def gather_pair(i, _):
        d0 = i * 2
        c0 = fire(d0, sem)
        c1 = fire(d0 + 1, sem2)
        for cp in c0:
            cp.wait()
        for cp in c1:
            cp.wait()
        return 0

    lax.fori_loop(0, LATENT_DIM // 2, gather_pair, 0, unroll=False)

    for cp in bias_copies:
        cp.wait()

    gbv = gb_v[...]

    def group(g, _):
        sl = pl.ds(g * 16, 16)
        acc = bcols[0, sl] * lcols[0, sl]
        for d in range(1, LATENT_DIM):
            acc = acc + bcols[d, sl] * lcols[d, sl]
        x = gbv + bb_v[sl] + lb_v[sl] + acc
        out_v[sl] = 1.0 / (1.0 + jnp.exp(-x))
        return 0

    lax.fori_loop(0, GROUPS, group, 0, unroll=False)

    pltpu.sync_copy(out_v, out_hbm.at[pl.ds(base, PAIRS_PER_WORKER)])


@jax.jit
def _predict(bids, lids, gb, leg_bias_t, bill_bias_t, leg_emb_t, bill_emb_t):
    leg_t = _dim_major_flat(leg_emb_t, leg_bias_t, LEG_BLK, 1, LEG_STRIDE)
    bill_t = _dim_major_flat(bill_emb_t, bill_bias_t, BILL_BLK, 4,
                             BILL_STRIDE)

    mesh = plsc.VectorSubcoreMesh(core_axis_name="c", subcore_axis_name="s")
    k = pl.kernel(
        _sc_body,
        out_type=jax.ShapeDtypeStruct((BATCH,), jnp.float32),
        mesh=mesh,
        compiler_params=pltpu.CompilerParams(needs_layout_passes=False,
                                             use_tc_tiling_on_sc=False),
        scratch_types=[
            pltpu.VMEM((PAIRS_PER_WORKER,), jnp.int32),
            pltpu.VMEM((PAIRS_PER_WORKER,), jnp.int32),
            pltpu.VMEM((ROWS, PAIRS_PER_WORKER), jnp.int32),
            pltpu.VMEM((ROWS, PAIRS_PER_WORKER), jnp.int32),
            pltpu.VMEM((LATENT_DIM, PAIRS_PER_WORKER), jnp.float32),
            pltpu.VMEM((LATENT_DIM, PAIRS_PER_WORKER), jnp.float32),
            pltpu.VMEM((PAIRS_PER_WORKER,), jnp.float32),
            pltpu.VMEM((PAIRS_PER_WORKER,), jnp.float32),
            pltpu.VMEM((16,), jnp.float32),
            pltpu.VMEM((PAIRS_PER_WORKER,), jnp.float32),
            pltpu.SemaphoreType.DMA,
            pltpu.SemaphoreType.DMA,
            pltpu.SemaphoreType.DMA,
        ],
    )
    return k(bids, lids, gb, leg_t, bill_t)


def kernel(bill_ids, legislator_ids, global_bias, legislator_bias, bill_bias,
           legislator_embedding, bill_embedding):
    bids = bill_ids.astype(jnp.int32)
    lids = legislator_ids.astype(jnp.int32)
    gb = jnp.broadcast_to(jnp.reshape(global_bias, (1,)), (16,))
    leg_bias_t = jnp.transpose(legislator_bias)
    bill_bias_t = jnp.transpose(bill_bias)
    leg_emb_t = jnp.transpose(legislator_embedding)
    bill_emb_t = jnp.transpose(bill_embedding)
    return _predict(bids, lids, gb, leg_bias_t, bill_bias_t,
                    leg_emb_t, bill_emb_t)


# speedup vs baseline: 2.4289x; 1.0398x over previous
import jax
import jax.numpy as jnp
from jax import lax
from jax.experimental import pallas as pl
from jax.experimental.pallas import tpu as pltpu
from jax.experimental.pallas import tpu_sc as plsc

NUM_BILLS = 1000000
NUM_LEGS = 100000
BILL_STRIDE = 1007616
LEG_STRIDE = 102400
BILL_BLK = BILL_STRIDE // 3
LEG_BLK = LEG_STRIDE // 1

BATCH = 16384
LATENT_DIM = 16
ROWS = LATENT_DIM + 1
NUM_WORKERS = 32
PAIRS_PER_WORKER = BATCH // NUM_WORKERS
CHUNK = 128
CHUNKS_PER_WORKER = PAIRS_PER_WORKER // CHUNK
GROUPS = PAIRS_PER_WORKER // 16


def _flatten_body(t_ref, b_ref, out_ref):
    d = pl.program_id(1)

    @pl.when(d < LATENT_DIM)
    def _():
        out_ref[...] = t_ref[d, :]

    @pl.when(d == LATENT_DIM)
    def _():
        out_ref[...] = b_ref[0, :]


def _dim_major_flat(table_t, bias_t, blk, blocks_per_row, stride):
    return pl.pallas_call(
        _flatten_body,
        grid=(blocks_per_row, ROWS),
        compiler_params=pltpu.CompilerParams(
            vmem_limit_bytes=120 * 1024 * 1024),
        in_specs=[
            pl.BlockSpec((LATENT_DIM, blk), lambda j, d: (0, j)),
            pl.BlockSpec((1, blk), lambda j, d: (0, j)),
        ],
        out_specs=pl.BlockSpec(
            (blk,), lambda j, d: (d * blocks_per_row + j,)),
        out_shape=jax.ShapeDtypeStruct((ROWS * stride,), jnp.float32),
    )(table_t, bias_t)


def _sc_body(bids, lids, gb, leg_t, bill_t, out_hbm,
             bidx, lidx, bgidx, lgidx, bcols, lcols, bb_v, lb_v, gb_v,
             out_v, sem, sem2, sem3):
    wid = lax.axis_index("s") * 2 + lax.axis_index("c")
    base = wid * PAIRS_PER_WORKER

    pltpu.sync_copy(bids.at[pl.ds(base, PAIRS_PER_WORKER)], bidx)
    pltpu.sync_copy(lids.at[pl.ds(base, PAIRS_PER_WORKER)], lidx)
    pltpu.sync_copy(gb, gb_v)

    def build(v, _):
        sl = pl.ds(v * 16, 16)
        bv = bidx[sl]
        lv = lidx[sl]
        for d in range(ROWS):
            bgidx[d, sl] = bv + d * BILL_STRIDE
            lgidx[d, sl] = lv + d * LEG_STRIDE
        return 0

    lax.fori_loop(0, GROUPS, build, 0, unroll=False)

    bias_copies = []
    for c in range(CHUNKS_PER_WORKER):
        sl = pl.ds(c * CHUNK, CHUNK)
        bias_copies.append(pltpu.async_copy(
            bill_t.at[bgidx.at[LATENT_DIM, sl]], bb_v.at[sl], sem3))
        bias_copies.append(pltpu.async_copy(
            leg_t.at[lgidx.at[LATENT_DIM, sl]], lb_v.at[sl], sem3))

    def fire(d, s):
        copies = []
        for c in range(CHUNKS_PER_WORKER):
            sl = pl.ds(c * CHUNK, CHUNK)
            copies.append(pltpu.async_copy(
                bill_t.at[bgidx.at[d, sl]], bcols.at[d, sl], s))
            copies.append(pltpu.async_copy(
                leg_t.at[lgidx.at[d, sl]], lcols.at[d, sl], s))
        return copies

    def gather_pair(i, _):
        d0 = i * 2
        c0 = fire(d0, sem)
        c1 = fire(d0 + 1, sem2)
        for cp in c0:
            cp.wait()
        for cp in c1:
            cp.wait()
        return 0

    lax.fori_loop(0, LATENT_DIM // 2, gather_pair, 0, unroll=False)

    for cp in bias_copies:
        cp.wait()

    gbv = gb_v[...]

    def group(g, _):
        sl = pl.ds(g * 16, 16)
        acc = bcols[0, sl] * lcols[0, sl]
        for d in range(1, LATENT_DIM):
            acc = acc + bcols[d, sl] * lcols[d, sl]
        x = gbv + bb_v[sl] + lb_v[sl] + acc
        out_v[sl] = 1.0 / (1.0 + jnp.exp(-x))
        return 0

    lax.fori_loop(0, GROUPS, group, 0, unroll=False)

    pltpu.sync_copy(out_v, out_hbm.at[pl.ds(base, PAIRS_PER_WORKER)])


@jax.jit
def _predict(bids, lids, gb, leg_bias_t, bill_bias_t, leg_emb_t, bill_emb_t):
    leg_t = _dim_major_flat(leg_emb_t, leg_bias_t, LEG_BLK, 1, LEG_STRIDE)
    bill_t = _dim_major_flat(bill_emb_t, bill_bias_t, BILL_BLK, 3,
                             BILL_STRIDE)

    mesh = plsc.VectorSubcoreMesh(core_axis_name="c", subcore_axis_name="s")
    k = pl.kernel(
        _sc_body,
        out_type=jax.ShapeDtypeStruct((BATCH,), jnp.float32),
        mesh=mesh,
        compiler_params=pltpu.CompilerParams(needs_layout_passes=False,
                                             use_tc_tiling_on_sc=False),
        scratch_types=[
            pltpu.VMEM((PAIRS_PER_WORKER,), jnp.int32),
            pltpu.VMEM((PAIRS_PER_WORKER,), jnp.int32),
            pltpu.VMEM((ROWS, PAIRS_PER_WORKER), jnp.int32),
            pltpu.VMEM((ROWS, PAIRS_PER_WORKER), jnp.int32),
            pltpu.VMEM((LATENT_DIM, PAIRS_PER_WORKER), jnp.float32),
            pltpu.VMEM((LATENT_DIM, PAIRS_PER_WORKER), jnp.float32),
            pltpu.VMEM((PAIRS_PER_WORKER,), jnp.float32),
            pltpu.VMEM((PAIRS_PER_WORKER,), jnp.float32),
            pltpu.VMEM((16,), jnp.float32),
            pltpu.VMEM((PAIRS_PER_WORKER,), jnp.float32),
            pltpu.SemaphoreType.DMA,
            pltpu.SemaphoreType.DMA,
            pltpu.SemaphoreType.DMA,
        ],
    )
    return k(bids, lids, gb, leg_t, bill_t)


def kernel(bill_ids, legislator_ids, global_bias, legislator_bias, bill_bias,
           legislator_embedding, bill_embedding):
    bids = bill_ids.astype(jnp.int32)
    lids = legislator_ids.astype(jnp.int32)
    gb = jnp.broadcast_to(jnp.reshape(global_bias, (1,)), (16,))
    leg_bias_t = jnp.transpose(legislator_bias)
    bill_bias_t = jnp.transpose(bill_bias)
    leg_emb_t = jnp.transpose(legislator_embedding)
    bill_emb_t = jnp.transpose(bill_embedding)
    return _predict(bids, lids, gb, leg_bias_t, bill_bias_t,
                    leg_emb_t, bill_emb_t)
